# baseline (speedup 1.0000x reference)
import jax
import jax.numpy as jnp
from jax.experimental import pallas as pl
from jax.experimental.pallas import tpu as pltpu


def _mm_tanh_kernel(neigh_ref, w_ref, out_ref):
    out_ref[...] = jnp.tanh(
        jax.lax.dot_general(
            neigh_ref[...], w_ref[...],
            dimension_numbers=(((1,), (0,)), ((), ())),
            preferred_element_type=jnp.float32,
        )
    )


def kernel(ent_emb, edge_index, neigh_w):
    N, D = ent_emb.shape
    src = edge_index[0]
    dst = edge_index[1]
    e = jnp.sum(ent_emb[src] * ent_emb[dst], axis=-1)
    m = jax.ops.segment_max(e, dst, num_segments=N)
    m = jnp.where(jnp.isfinite(m), m, 0.0)
    ex = jnp.exp(e - m[dst])
    s = jax.ops.segment_sum(ex, dst, num_segments=N)
    alpha = ex / s[dst]
    msg = ent_emb[src] * alpha[:, None]
    neigh = jax.ops.segment_sum(msg, dst, num_segments=N)

    blk = 400
    out = pl.pallas_call(
        _mm_tanh_kernel,
        out_shape=jax.ShapeDtypeStruct((N, D), jnp.float32),
        grid=(N // blk,),
        in_specs=[
            pl.BlockSpec((blk, D), lambda i: (i, 0)),
            pl.BlockSpec((D, D), lambda i: (0, 0)),
        ],
        out_specs=pl.BlockSpec((blk, D), lambda i: (i, 0)),
    )(neigh, neigh_w)
    return out


# speedup vs baseline: 10.9093x; 10.9093x over previous
import dataclasses
import functools

import jax
import jax.numpy as jnp
from jax import lax
from jax.experimental import pallas as pl
from jax.experimental.pallas import tpu as pltpu
from jax.experimental.pallas import tpu_sc as plsc

N, E, D = 10000, 320000, 128
NC, NS, L = 2, 16, 16
NW = NC * NS
ET = E // NW
C = 80
NCH = ET // C
NEG = -1e30
NACC = 10240
ROWS_PER_TILE = NACC // NS

_mesh = plsc.VectorSubcoreMesh(core_axis_name="c", subcore_axis_name="s")

_sc_params = pltpu.CompilerParams()
if "needs_layout_passes" in pltpu.CompilerParams.__dataclass_fields__:
    _sc_params = dataclasses.replace(_sc_params, needs_layout_passes=False)


def _phase1(ent_emb, src_arr, dst_arr):
    @functools.partial(
        pl.kernel,
        out_type=(
            jax.ShapeDtypeStruct((E,), jnp.float32),
            jax.ShapeDtypeStruct((NW * N,), jnp.float32),
            jax.ShapeDtypeStruct((NW * N,), jnp.float32),
        ),
        mesh=_mesh,
        compiler_params=_sc_params,
        scratch_types=[
            pltpu.VMEM((ET,), jnp.int32),
            pltpu.VMEM((ET,), jnp.int32),
            pltpu.VMEM((ET,), jnp.float32),
            pltpu.VMEM((N,), jnp.float32),
            pltpu.VMEM((N,), jnp.float32),
            pltpu.VMEM((C, D), jnp.float32),
            pltpu.VMEM((C, D), jnp.float32),
            pltpu.SemaphoreType.DMA,
            pltpu.SemaphoreType.DMA,
        ],
    )
    def k(emb_hbm, srca_hbm, dsta_hbm, e_hbm, m_hbm, s_hbm,
          src_v, dst_v, e_v, m_v, s_v, rs_v, rd_v, sem0, sem1):
        wid = lax.axis_index("s") * NC + lax.axis_index("c")
        base = wid * ET
        pltpu.sync_copy(srca_hbm.at[pl.ds(base, ET)], src_v)
        pltpu.sync_copy(dsta_hbm.at[pl.ds(base, ET)], dst_v)

        neg16 = jnp.full((L,), NEG, jnp.float32)
        zero16 = jnp.zeros((L,), jnp.float32)

        @pl.loop(0, N, step=L)
        def _init(i):
            m_v[pl.ds(i, L)] = neg16
            s_v[pl.ds(i, L)] = zero16

        @pl.loop(0, ET, step=C)
        def _chunk(c0):
            pltpu.async_copy(emb_hbm.at[src_v.at[pl.ds(c0, C)]], rs_v, sem0)
            pltpu.async_copy(emb_hbm.at[dst_v.at[pl.ds(c0, C)]], rd_v, sem1)
            pltpu.make_async_copy(emb_hbm.at[src_v.at[pl.ds(c0, C)]], rs_v, sem0).wait()
            pltpu.make_async_copy(emb_hbm.at[dst_v.at[pl.ds(c0, C)]], rd_v, sem1).wait()

            lane_iota = lax.broadcasted_iota(jnp.int32, (L,), 0)

            @pl.loop(0, C, step=L)
            def _grp(g):
                def _dot(jj, evec):
                    j = g + jj
                    acc = rs_v[j, pl.ds(0, L)] * rd_v[j, pl.ds(0, L)]
                    for kk in range(1, D // L):
                        acc = acc + (rs_v[j, pl.ds(kk * L, L)]
                                     * rd_v[j, pl.ds(kk * L, L)])
                    return jnp.where(lane_iota == jj, jnp.sum(acc), evec)

                e16 = lax.fori_loop(0, L, _dot, jnp.zeros((L,), jnp.float32))
                e_v[pl.ds(c0 + g, L)] = e16
                dst16 = dst_v[pl.ds(c0 + g, L)]

                def _cond(_):
                    cur = plsc.load_gather(m_v, [dst16])
                    return jnp.any(cur < e16)

                def _body(it):
                    cur = plsc.load_gather(m_v, [dst16])
                    plsc.store_scatter(m_v, [dst16], jnp.maximum(cur, e16),
                                       mask=cur < e16)
                    return it + 1

                lax.while_loop(_cond, _body, 0)

        @pl.loop(0, ET, step=L)
        def _sgrp(g):
            dst16 = dst_v[pl.ds(g, L)]
            e16 = e_v[pl.ds(g, L)]
            mv = plsc.load_gather(m_v, [dst16])
            plsc.addupdate_scatter(s_v, [dst16], jnp.exp(e16 - mv))

        pltpu.sync_copy(e_v, e_hbm.at[pl.ds(base, ET)])
        pltpu.sync_copy(m_v, m_hbm.at[pl.ds(wid * N, N)])
        pltpu.sync_copy(s_v, s_hbm.at[pl.ds(wid * N, N)])

    return k(ent_emb, src_arr, dst_arr)


def _combine_kernel(m_ref, s_ref, ms_ref):
    m = jnp.max(m_ref[...], axis=0)
    s = jnp.sum(s_ref[...] * jnp.exp(m_ref[...] - m[None, :]), axis=0)
    inv_s = jnp.where(s > 0.0, 1.0 / s, 0.0)
    ms_ref[...] = jnp.stack([m, inv_s], axis=0)


def _phase3(ent_emb, src_arr, dst_arr, e_all, ms_flat):
    @functools.partial(
        pl.kernel,
        out_type=jax.ShapeDtypeStruct((NC, NACC, D), jnp.float32),
        mesh=_mesh,
        compiler_params=_sc_params,
        scratch_types=[
            pltpu.VMEM((N,), jnp.float32),
            pltpu.VMEM((N,), jnp.float32),
            pltpu.VMEM((C,), jnp.int32),
            pltpu.VMEM((C,), jnp.int32),
            pltpu.VMEM((C,), jnp.float32),
            pltpu.VMEM((C, D), jnp.float32),
            pltpu.VMEM_SHARED((NACC, D), jnp.float32),
            pltpu.SemaphoreType.DMA,
        ],
    )
    def k(emb_hbm, srca_hbm, dsta_hbm, e_hbm, ms_hbm, acc_hbm,
          m_v, is_v, csrc_v, cdst_v, ce_v, rows_v, shared, sem0):
        core = lax.axis_index("c")
        sid = lax.axis_index("s")
        wid = sid * NC + core
        base = wid * ET
        pltpu.sync_copy(ms_hbm.at[pl.ds(0, N)], m_v)
        pltpu.sync_copy(ms_hbm.at[pl.ds(N, N)], is_v)

        zero16 = jnp.zeros((L,), jnp.float32)

        @pl.loop(0, C * D, step=L)
        def _z(i):
            rows_v[i // D, pl.ds((i % D) // L * L, L)] = zero16

        zbase = sid * ROWS_PER_TILE
        for t in range(0, ROWS_PER_TILE, C):
            pltpu.sync_copy(rows_v, shared.at[pl.ds(zbase + t, C)])
        plsc.subcore_barrier()

        @pl.loop(0, ET, step=C)
        def _chunk(c0):
            pltpu.sync_copy(srca_hbm.at[pl.ds(base + c0, C)], csrc_v)
            pltpu.async_copy(emb_hbm.at[csrc_v], rows_v, sem0)
            pltpu.sync_copy(dsta_hbm.at[pl.ds(base + c0, C)], cdst_v)
            pltpu.sync_copy(e_hbm.at[pl.ds(base + c0, C)], ce_v)
            pltpu.make_async_copy(emb_hbm.at[csrc_v], rows_v, sem0).wait()

            @pl.loop(0, C, step=L)
            def _grp(g):
                dst16 = cdst_v[pl.ds(g, L)]
                e16 = ce_v[pl.ds(g, L)]
                mv = plsc.load_gather(m_v, [dst16])
                iv = plsc.load_gather(is_v, [dst16])
                a16 = jnp.exp(e16 - mv) * iv
                for jj in range(L):
                    aj = jnp.full((L,), a16[jj], jnp.float32)
                    for kk in range(D // L):
                        rows_v[g + jj, pl.ds(kk * L, L)] = (
                            rows_v[g + jj, pl.ds(kk * L, L)] * aj)

            pltpu.sync_copy(rows_v, shared.at[cdst_v], add=True)

        plsc.subcore_barrier()
        pltpu.sync_copy(shared.at[pl.ds(sid * ROWS_PER_TILE, ROWS_PER_TILE)],
                        acc_hbm.at[core, pl.ds(sid * ROWS_PER_TILE, ROWS_PER_TILE)])

    return k(ent_emb, src_arr, dst_arr, e_all, ms_flat)


def _mm_tanh_kernel(p_ref, w_ref, o_ref):
    x = p_ref[0] + p_ref[1]
    o_ref[...] = jnp.tanh(
        lax.dot_general(x, w_ref[...], (((1,), (0,)), ((), ())),
                        preferred_element_type=jnp.float32))


def kernel(ent_emb, edge_index, neigh_w):
    assert ent_emb.shape == (N, D) and edge_index.shape == (2, E)

    src_arr = edge_index[0]
    dst_arr = edge_index[1]
    e_all, m_loc, s_loc = _phase1(ent_emb, src_arr, dst_arr)

    ms = pl.pallas_call(
        _combine_kernel,
        out_shape=jax.ShapeDtypeStruct((2, N), jnp.float32),
    )(m_loc.reshape(NW, N), s_loc.reshape(NW, N))

    partial = _phase3(ent_emb, src_arr, dst_arr, e_all, ms.reshape(2 * N))

    blk = 400
    out = pl.pallas_call(
        _mm_tanh_kernel,
        grid=(N // blk,),
        in_specs=[
            pl.BlockSpec((NC, blk, D), lambda i: (0, i, 0)),
            pl.BlockSpec((D, D), lambda i: (0, 0)),
        ],
        out_specs=pl.BlockSpec((blk, D), lambda i: (i, 0)),
        out_shape=jax.ShapeDtypeStruct((N, D), jnp.float32),
    )(partial, neigh_w)
    return out
